# baseline (speedup 1.0000x reference)
import functools

import jax
import jax.numpy as jnp
from jax import lax
from jax.experimental import pallas as pl
from jax.experimental.pallas import tpu as pltpu
from jax.experimental.pallas import tpu_sc as plsc

_HIDDEN = 768
_N_EXPERTS = 64
_TOP_K = 2
_D_FF = 384
_N_TOKENS = 2048
_N_SLOTS = _N_TOKENS * _TOP_K

_BLK = 128
_NB = _N_SLOTS // _BLK + _N_EXPERTS
_P = _NB * _BLK

_NW = 32
_LANES = 16


def _routing_body(hs_ref, gw_ref, w1_ref, w2_ref, p0_ref, p1_ref,
                  brow_ref, bexp_ref, bact_ref):
    logits = jnp.dot(hs_ref[...], gw_ref[...],
                     preferred_element_type=jnp.float32)
    col = lax.broadcasted_iota(jnp.int32, logits.shape, 1)
    m1 = jnp.max(logits, axis=1, keepdims=True)
    i1 = jnp.min(jnp.where(logits == m1, col, _N_EXPERTS), axis=1)
    masked = jnp.where(col == i1[:, None], -jnp.inf, logits)
    m2 = jnp.max(masked, axis=1, keepdims=True)
    i2 = jnp.min(jnp.where(masked == m2, col, _N_EXPERTS), axis=1)
    e2 = jnp.exp(m2[:, 0] - m1[:, 0])
    w1_ref[...] = 1.0 / (1.0 + e2)
    w2_ref[...] = e2 / (1.0 + e2)

    occ1 = (col == i1[:, None]).astype(jnp.int32)
    occ2 = (col == i2[:, None]).astype(jnp.int32)
    occ = occ1 + occ2
    inc = occ
    k = 1
    while k < _N_TOKENS:
        shifted = jnp.concatenate(
            [jnp.zeros((k, _N_EXPERTS), jnp.int32), inc[:-k, :]], axis=0)
        inc = inc + shifted
        k *= 2
    exc = inc - occ
    counts = jnp.sum(occ, axis=0, keepdims=True)
    padded = ((counts + _BLK - 1) // _BLK) * _BLK
    cum_p = padded
    k = 1
    while k < _N_EXPERTS:
        shifted = jnp.concatenate(
            [jnp.zeros((1, k), jnp.int32), cum_p[:, :-k]], axis=1)
        cum_p = cum_p + shifted
        k *= 2
    pstart = cum_p - padded
    pos_base1 = jnp.sum((pstart + exc) * occ1, axis=1)
    pos_base2 = jnp.sum((pstart + exc + occ1) * occ2, axis=1)
    p0_ref[...] = pos_base1
    p1_ref[...] = pos_base2

    nact = jnp.sum(padded) // _BLK
    bidx = lax.broadcasted_iota(jnp.int32, (_NB,), 0)
    brow = jnp.minimum(bidx, nact - 1)
    bexp = jnp.sum(
        (cum_p <= brow[:, None] * _BLK).astype(jnp.int32), axis=1)
    brow_ref[...] = brow
    bexp_ref[...] = jnp.clip(bexp, 0, _N_EXPERTS - 1)
    bact_ref[...] = (bidx < nact).astype(jnp.int32)


def _routing(hidden_states, gate_w):
    return pl.pallas_call(
        _routing_body,
        out_shape=[
            jax.ShapeDtypeStruct((_N_TOKENS,), jnp.float32),
            jax.ShapeDtypeStruct((_N_TOKENS,), jnp.float32),
            jax.ShapeDtypeStruct((_N_TOKENS,), jnp.int32),
            jax.ShapeDtypeStruct((_N_TOKENS,), jnp.int32),
            jax.ShapeDtypeStruct((_NB,), jnp.int32),
            jax.ShapeDtypeStruct((_NB,), jnp.int32),
            jax.ShapeDtypeStruct((_NB,), jnp.int32),
        ],
    )(hidden_states, gate_w)


_TOKW = _N_TOKENS // _NW


def _dispatch_body(hid_hbm, p0_hbm, p1_hbm, x_hbm, i0_v, i1_v, rows_v,
                   lsem, sem0, sem1):
    wid = lax.axis_index("s") * 2 + lax.axis_index("c")
    base = wid * _TOKW
    pltpu.sync_copy(p0_hbm.at[wid], i0_v)
    pltpu.sync_copy(p1_hbm.at[wid], i1_v)
    pltpu.async_copy(hid_hbm.at[pl.ds(base, _TOKW)], rows_v, lsem).wait()
    c0 = pltpu.make_async_copy(rows_v, x_hbm.at[i0_v], sem0)
    c1 = pltpu.make_async_copy(rows_v, x_hbm.at[i1_v], sem1)
    c0.start()
    c1.start()
    c0.wait()
    c1.wait()


def _dispatch(hidden_states, p0r, p1r):
    mesh = plsc.VectorSubcoreMesh(core_axis_name="c", subcore_axis_name="s")
    return pl.kernel(
        _dispatch_body,
        out_type=jax.ShapeDtypeStruct((_P, _HIDDEN), jnp.float32),
        mesh=mesh,
        scratch_types=[
            pltpu.VMEM((_TOKW,), jnp.int32),
            pltpu.VMEM((_TOKW,), jnp.int32),
            pltpu.VMEM((_TOKW, _HIDDEN), jnp.float32),
            pltpu.SemaphoreType.DMA,
            pltpu.SemaphoreType.DMA,
            pltpu.SemaphoreType.DMA,
        ],
    )(hidden_states, p0r, p1r)


def _ffn_body(brow, bexp, bact, x_ref, wg_ref, wu_ref, wd_ref, y_ref):
    i = pl.program_id(0)

    @pl.when(bact[i] == 1)
    def _():
        x = x_ref[...]
        g = jnp.dot(x, wg_ref[0], preferred_element_type=jnp.float32,
                    precision=lax.Precision.DEFAULT)
        u = jnp.dot(x, wu_ref[0], preferred_element_type=jnp.float32,
                    precision=lax.Precision.DEFAULT)
        inter = (g * jax.nn.sigmoid(g)) * u
        y_ref[...] = jnp.dot(inter, wd_ref[0], preferred_element_type=jnp.float32,
                             precision=lax.Precision.DEFAULT)


def _ffn(x_sorted, w_gate, w_up, w_down, brow, bexp, bact):
    grid_spec = pltpu.PrefetchScalarGridSpec(
        num_scalar_prefetch=3,
        grid=(_NB,),
        in_specs=[
            pl.BlockSpec((_BLK, _HIDDEN), lambda i, br, be, ba: (br[i], 0)),
            pl.BlockSpec((1, _HIDDEN, _D_FF), lambda i, br, be, ba: (be[i], 0, 0)),
            pl.BlockSpec((1, _HIDDEN, _D_FF), lambda i, br, be, ba: (be[i], 0, 0)),
            pl.BlockSpec((1, _D_FF, _HIDDEN), lambda i, br, be, ba: (be[i], 0, 0)),
        ],
        out_specs=pl.BlockSpec((_BLK, _HIDDEN), lambda i, br, be, ba: (br[i], 0)),
    )
    return pl.pallas_call(
        _ffn_body,
        grid_spec=grid_spec,
        out_shape=jax.ShapeDtypeStruct((_P, _HIDDEN), jnp.float32),
    )(brow, bexp, bact, x_sorted, w_gate, w_up, w_down)


def _lane_bcast(vec, t):
    lane = jnp.full((_LANES, 1), t, jnp.int32)
    return lax.gather(
        vec, lane,
        lax.GatherDimensionNumbers(
            offset_dims=(), collapsed_slice_dims=(0,), start_index_map=(0,)),
        slice_sizes=(1,),
        mode=lax.GatherScatterMode.PROMISE_IN_BOUNDS)


_HTOK = _TOKW // 2


def _combine_body(y_hbm, p0_hbm, p1_hbm, w0_hbm, w1_hbm, out_hbm,
                  i0_v, i1_v, w0_v, w1_v,
                  ra0, ra1, rb0, rb1, gsa, gsb, wsem):
    wid = lax.axis_index("s") * 2 + lax.axis_index("c")
    base = wid * _TOKW
    pltpu.sync_copy(p0_hbm.at[wid], i0_v)
    pltpu.sync_copy(p1_hbm.at[wid], i1_v)
    pltpu.sync_copy(w0_hbm.at[wid], w0_v)
    pltpu.sync_copy(w1_hbm.at[wid], w1_v)

    halves = (
        (0, i0_v.at[0], i1_v.at[0], ra0, ra1, gsa),
        (1, i0_v.at[1], i1_v.at[1], rb0, rb1, gsb),
    )
    copies = []
    for h, idx0, idx1, r0, r1, sem in halves:
        g0 = pltpu.make_async_copy(y_hbm.at[idx0], r0, sem)
        g1 = pltpu.make_async_copy(y_hbm.at[idx1], r1, sem)
        g0.start()
        g1.start()
        copies.append((g0, g1))

    writes = []
    for h, idx0, idx1, r0, r1, sem in halves:
        g0, g1 = copies[h]
        g0.wait()
        g1.wait()

        def wadd_group(g, _, r0=r0, r1=r1, h=h):
            w0c = w0_v[h, pl.ds(g * _LANES, _LANES)]
            w1c = w1_v[h, pl.ds(g * _LANES, _LANES)]
            for t in range(_LANES):
                tok = g * _LANES + t
                w0b = _lane_bcast(w0c, t)
                w1b = _lane_bcast(w1c, t)
                for c in range(_HIDDEN // _LANES):
                    sl = pl.ds(c * _LANES, _LANES)
                    r0[tok, sl] = r0[tok, sl] * w0b + r1[tok, sl] * w1b
            return 0

        lax.fori_loop(0, _HTOK // _LANES, wadd_group, 0)
        w = pltpu.make_async_copy(
            r0, out_hbm.at[pl.ds(base + h * _HTOK, _HTOK)], wsem)
        w.start()
        writes.append(w)
    for w in writes:
        w.wait()


def _combine(y_sorted, p0r, p1r, w0r, w1r):
    mesh = plsc.VectorSubcoreMesh(core_axis_name="c", subcore_axis_name="s")
    return pl.kernel(
        _combine_body,
        out_type=jax.ShapeDtypeStruct((_N_TOKENS, _HIDDEN), jnp.float32),
        mesh=mesh,
        scratch_types=[
            pltpu.VMEM((2, _HTOK), jnp.int32),
            pltpu.VMEM((2, _HTOK), jnp.int32),
            pltpu.VMEM((2, _HTOK), jnp.float32),
            pltpu.VMEM((2, _HTOK), jnp.float32),
            pltpu.VMEM((_HTOK, _HIDDEN), jnp.float32),
            pltpu.VMEM((_HTOK, _HIDDEN), jnp.float32),
            pltpu.VMEM((_HTOK, _HIDDEN), jnp.float32),
            pltpu.VMEM((_HTOK, _HIDDEN), jnp.float32),
            pltpu.SemaphoreType.DMA,
            pltpu.SemaphoreType.DMA,
            pltpu.SemaphoreType.DMA,
        ],
    )(y_sorted, p0r, p1r, w0r, w1r)


@jax.jit
def kernel(hidden_states, gate_w, w_gate, w_up, w_down):
    w1, w2, p0, p1, brow, bexp, bact = _routing(hidden_states, gate_w)

    p0r = p0.reshape(_NW, _TOKW)
    p1r = p1.reshape(_NW, _TOKW)
    p0h = p0.reshape(_NW, 2, _HTOK)
    p1h = p1.reshape(_NW, 2, _HTOK)
    w0h = w1.reshape(_NW, 2, _HTOK)
    w1h = w2.reshape(_NW, 2, _HTOK)

    x_sorted = _dispatch(hidden_states, p0r, p1r)
    y_sorted = _ffn(x_sorted, w_gate, w_up, w_down, brow, bexp, bact)
    return _combine(y_sorted, p0h, p1h, w0h, w1h)


# speedup vs baseline: 1.0032x; 1.0032x over previous
import functools

import jax
import jax.numpy as jnp
from jax import lax
from jax.experimental import pallas as pl
from jax.experimental.pallas import tpu as pltpu
from jax.experimental.pallas import tpu_sc as plsc

_HIDDEN = 768
_N_EXPERTS = 64
_TOP_K = 2
_D_FF = 384
_N_TOKENS = 2048
_N_SLOTS = _N_TOKENS * _TOP_K

_BLK = 128
_NB = _N_SLOTS // _BLK + _N_EXPERTS
_P = _NB * _BLK

_NW = 32
_LANES = 16


def _routing_body(hs_ref, gw_ref, w1_ref, w2_ref, p0_ref, p1_ref,
                  brow_ref, bexp_ref, bact_ref):
    logits = jnp.dot(hs_ref[...], gw_ref[...],
                     preferred_element_type=jnp.float32)
    col = lax.broadcasted_iota(jnp.int32, logits.shape, 1)
    m1 = jnp.max(logits, axis=1, keepdims=True)
    i1 = jnp.min(jnp.where(logits == m1, col, _N_EXPERTS), axis=1)
    masked = jnp.where(col == i1[:, None], -jnp.inf, logits)
    m2 = jnp.max(masked, axis=1, keepdims=True)
    i2 = jnp.min(jnp.where(masked == m2, col, _N_EXPERTS), axis=1)
    e2 = jnp.exp(m2[:, 0] - m1[:, 0])
    w1_ref[...] = 1.0 / (1.0 + e2)
    w2_ref[...] = e2 / (1.0 + e2)

    occ1 = (col == i1[:, None]).astype(jnp.int32)
    occ2 = (col == i2[:, None]).astype(jnp.int32)
    occ = occ1 + occ2
    inc = occ
    k = 1
    while k < _N_TOKENS:
        shifted = jnp.concatenate(
            [jnp.zeros((k, _N_EXPERTS), jnp.int32), inc[:-k, :]], axis=0)
        inc = inc + shifted
        k *= 2
    exc = inc - occ
    counts = jnp.sum(occ, axis=0, keepdims=True)
    padded = ((counts + _BLK - 1) // _BLK) * _BLK
    cum_p = padded
    k = 1
    while k < _N_EXPERTS:
        shifted = jnp.concatenate(
            [jnp.zeros((1, k), jnp.int32), cum_p[:, :-k]], axis=1)
        cum_p = cum_p + shifted
        k *= 2
    pstart = cum_p - padded
    pos_base1 = jnp.sum((pstart + exc) * occ1, axis=1)
    pos_base2 = jnp.sum((pstart + exc + occ1) * occ2, axis=1)
    p0_ref[...] = pos_base1
    p1_ref[...] = pos_base2

    nact = jnp.sum(padded) // _BLK
    bidx = lax.broadcasted_iota(jnp.int32, (_NB,), 0)
    brow = jnp.minimum(bidx, nact - 1)
    bexp = jnp.sum(
        (cum_p <= brow[:, None] * _BLK).astype(jnp.int32), axis=1)
    brow_ref[...] = brow
    bexp_ref[...] = jnp.clip(bexp, 0, _N_EXPERTS - 1)
    bact_ref[...] = (bidx < nact).astype(jnp.int32)


def _routing(hidden_states, gate_w):
    return pl.pallas_call(
        _routing_body,
        out_shape=[
            jax.ShapeDtypeStruct((_N_TOKENS,), jnp.float32),
            jax.ShapeDtypeStruct((_N_TOKENS,), jnp.float32),
            jax.ShapeDtypeStruct((_N_TOKENS,), jnp.int32),
            jax.ShapeDtypeStruct((_N_TOKENS,), jnp.int32),
            jax.ShapeDtypeStruct((_NB,), jnp.int32),
            jax.ShapeDtypeStruct((_NB,), jnp.int32),
            jax.ShapeDtypeStruct((_NB,), jnp.int32),
        ],
    )(hidden_states, gate_w)


_TOKW = _N_TOKENS // _NW


def _dispatch_body(hid_hbm, p0_hbm, p1_hbm, x_hbm, i0_v, i1_v, rows_v,
                   lsem, sem0, sem1):
    wid = lax.axis_index("s") * 2 + lax.axis_index("c")
    base = wid * _TOKW
    pltpu.sync_copy(p0_hbm.at[wid], i0_v)
    pltpu.sync_copy(p1_hbm.at[wid], i1_v)
    pltpu.async_copy(hid_hbm.at[pl.ds(base, _TOKW)], rows_v, lsem).wait()
    c0 = pltpu.make_async_copy(rows_v, x_hbm.at[i0_v], sem0)
    c1 = pltpu.make_async_copy(rows_v, x_hbm.at[i1_v], sem1)
    c0.start()
    c1.start()
    c0.wait()
    c1.wait()


def _dispatch(hidden_states, p0r, p1r):
    mesh = plsc.VectorSubcoreMesh(core_axis_name="c", subcore_axis_name="s")
    return pl.kernel(
        _dispatch_body,
        out_type=jax.ShapeDtypeStruct((_P, _HIDDEN), jnp.float32),
        mesh=mesh,
        scratch_types=[
            pltpu.VMEM((_TOKW,), jnp.int32),
            pltpu.VMEM((_TOKW,), jnp.int32),
            pltpu.VMEM((_TOKW, _HIDDEN), jnp.float32),
            pltpu.SemaphoreType.DMA,
            pltpu.SemaphoreType.DMA,
            pltpu.SemaphoreType.DMA,
        ],
    )(hidden_states, p0r, p1r)


def _ffn_body(brow, bexp, bact, x_ref, wg_ref, wu_ref, wd_ref, y_ref):
    i = pl.program_id(0)

    @pl.when(bact[i] == 1)
    def _():
        x = x_ref[...]
        g = jnp.dot(x, wg_ref[0], preferred_element_type=jnp.float32)
        u = jnp.dot(x, wu_ref[0], preferred_element_type=jnp.float32)
        inter = (g * jax.nn.sigmoid(g)) * u
        y_ref[...] = jnp.dot(inter, wd_ref[0], preferred_element_type=jnp.float32)


def _ffn(x_sorted, w_gate, w_up, w_down, brow, bexp, bact):
    grid_spec = pltpu.PrefetchScalarGridSpec(
        num_scalar_prefetch=3,
        grid=(_NB,),
        in_specs=[
            pl.BlockSpec((_BLK, _HIDDEN), lambda i, br, be, ba: (br[i], 0)),
            pl.BlockSpec((1, _HIDDEN, _D_FF), lambda i, br, be, ba: (be[i], 0, 0)),
            pl.BlockSpec((1, _HIDDEN, _D_FF), lambda i, br, be, ba: (be[i], 0, 0)),
            pl.BlockSpec((1, _D_FF, _HIDDEN), lambda i, br, be, ba: (be[i], 0, 0)),
        ],
        out_specs=pl.BlockSpec((_BLK, _HIDDEN), lambda i, br, be, ba: (br[i], 0)),
    )
    return pl.pallas_call(
        _ffn_body,
        grid_spec=grid_spec,
        out_shape=jax.ShapeDtypeStruct((_P, _HIDDEN), jnp.float32),
    )(brow, bexp, bact, x_sorted, w_gate, w_up, w_down)


def _lane_bcast(vec, t):
    lane = jnp.full((_LANES, 1), t, jnp.int32)
    return lax.gather(
        vec, lane,
        lax.GatherDimensionNumbers(
            offset_dims=(), collapsed_slice_dims=(0,), start_index_map=(0,)),
        slice_sizes=(1,),
        mode=lax.GatherScatterMode.PROMISE_IN_BOUNDS)


_HTOK = _TOKW // 2


def _combine_body(y_hbm, p0_hbm, p1_hbm, w0_hbm, w1_hbm, out_hbm,
                  i0_v, i1_v, w0_v, w1_v,
                  ra0, ra1, rb0, rb1, gsa, gsb, wsem):
    wid = lax.axis_index("s") * 2 + lax.axis_index("c")
    base = wid * _TOKW
    pltpu.sync_copy(p0_hbm.at[wid], i0_v)
    pltpu.sync_copy(p1_hbm.at[wid], i1_v)
    pltpu.sync_copy(w0_hbm.at[wid], w0_v)
    pltpu.sync_copy(w1_hbm.at[wid], w1_v)

    halves = (
        (0, i0_v.at[0], i1_v.at[0], ra0, ra1, gsa),
        (1, i0_v.at[1], i1_v.at[1], rb0, rb1, gsb),
    )
    copies = []
    for h, idx0, idx1, r0, r1, sem in halves:
        g0 = pltpu.make_async_copy(y_hbm.at[idx0], r0, sem)
        g1 = pltpu.make_async_copy(y_hbm.at[idx1], r1, sem)
        g0.start()
        g1.start()
        copies.append((g0, g1))

    writes = []
    for h, idx0, idx1, r0, r1, sem in halves:
        g0, g1 = copies[h]
        g0.wait()
        g1.wait()

        def wadd_group(g, _, r0=r0, r1=r1, h=h):
            w0c = w0_v[h, pl.ds(g * _LANES, _LANES)]
            w1c = w1_v[h, pl.ds(g * _LANES, _LANES)]
            for t in range(_LANES):
                tok = g * _LANES + t
                w0b = _lane_bcast(w0c, t)
                w1b = _lane_bcast(w1c, t)
                for c in range(_HIDDEN // _LANES):
                    sl = pl.ds(c * _LANES, _LANES)
                    r0[tok, sl] = r0[tok, sl] * w0b + r1[tok, sl] * w1b
            return 0

        lax.fori_loop(0, _HTOK // _LANES, wadd_group, 0)
        w = pltpu.make_async_copy(
            r0, out_hbm.at[pl.ds(base + h * _HTOK, _HTOK)], wsem)
        w.start()
        writes.append(w)
    for w in writes:
        w.wait()


def _combine(y_sorted, p0r, p1r, w0r, w1r):
    mesh = plsc.VectorSubcoreMesh(core_axis_name="c", subcore_axis_name="s")
    return pl.kernel(
        _combine_body,
        out_type=jax.ShapeDtypeStruct((_N_TOKENS, _HIDDEN), jnp.float32),
        mesh=mesh,
        scratch_types=[
            pltpu.VMEM((2, _HTOK), jnp.int32),
            pltpu.VMEM((2, _HTOK), jnp.int32),
            pltpu.VMEM((2, _HTOK), jnp.float32),
            pltpu.VMEM((2, _HTOK), jnp.float32),
            pltpu.VMEM((_HTOK, _HIDDEN), jnp.float32),
            pltpu.VMEM((_HTOK, _HIDDEN), jnp.float32),
            pltpu.VMEM((_HTOK, _HIDDEN), jnp.float32),
            pltpu.VMEM((_HTOK, _HIDDEN), jnp.float32),
            pltpu.SemaphoreType.DMA,
            pltpu.SemaphoreType.DMA,
            pltpu.SemaphoreType.DMA,
        ],
    )(y_sorted, p0r, p1r, w0r, w1r)


@jax.jit
def kernel(hidden_states, gate_w, w_gate, w_up, w_down):
    w1, w2, p0, p1, brow, bexp, bact = _routing(hidden_states, gate_w)

    p0r = p0.reshape(_NW, _TOKW)
    p1r = p1.reshape(_NW, _TOKW)
    p0h = p0.reshape(_NW, 2, _HTOK)
    p1h = p1.reshape(_NW, 2, _HTOK)
    w0h = w1.reshape(_NW, 2, _HTOK)
    w1h = w2.reshape(_NW, 2, _HTOK)

    x_sorted = _dispatch(hidden_states, p0r, p1r)
    y_sorted = _ffn(x_sorted, w_gate, w_up, w_down, brow, bexp, bact)
    return _combine(y_sorted, p0h, p1h, w0h, w1h)


# speedup vs baseline: 1.0356x; 1.0323x over previous
import functools

import jax
import jax.numpy as jnp
from jax import lax
from jax.experimental import pallas as pl
from jax.experimental.pallas import tpu as pltpu
from jax.experimental.pallas import tpu_sc as plsc

_HIDDEN = 768
_N_EXPERTS = 64
_TOP_K = 2
_D_FF = 384
_N_TOKENS = 2048
_N_SLOTS = _N_TOKENS * _TOP_K

_BLK = 128
_NB = _N_SLOTS // _BLK + _N_EXPERTS
_P = _NB * _BLK

_NW = 32
_LANES = 16


def _routing_body(hs_ref, gw_ref, w1_ref, w2_ref, p0_ref, p1_ref,
                  brow_ref, bexp_ref, bact_ref):
    logits = jnp.dot(hs_ref[...], gw_ref[...],
                     preferred_element_type=jnp.float32)
    col = lax.broadcasted_iota(jnp.int32, logits.shape, 1)
    m1 = jnp.max(logits, axis=1, keepdims=True)
    i1 = jnp.min(jnp.where(logits == m1, col, _N_EXPERTS), axis=1)
    masked = jnp.where(col == i1[:, None], -jnp.inf, logits)
    m2 = jnp.max(masked, axis=1, keepdims=True)
    i2 = jnp.min(jnp.where(masked == m2, col, _N_EXPERTS), axis=1)
    e2 = jnp.exp(m2[:, 0] - m1[:, 0])
    w1_ref[...] = 1.0 / (1.0 + e2)
    w2_ref[...] = e2 / (1.0 + e2)

    occ1 = (col == i1[:, None]).astype(jnp.int32)
    occ2 = (col == i2[:, None]).astype(jnp.int32)
    occ = occ1 + occ2
    inc = occ
    k = 1
    while k < _N_TOKENS:
        shifted = jnp.concatenate(
            [jnp.zeros((k, _N_EXPERTS), jnp.int32), inc[:-k, :]], axis=0)
        inc = inc + shifted
        k *= 2
    exc = inc - occ
    counts = jnp.sum(occ, axis=0, keepdims=True)
    padded = ((counts + _BLK - 1) // _BLK) * _BLK
    cum_p = padded
    k = 1
    while k < _N_EXPERTS:
        shifted = jnp.concatenate(
            [jnp.zeros((1, k), jnp.int32), cum_p[:, :-k]], axis=1)
        cum_p = cum_p + shifted
        k *= 2
    pstart = cum_p - padded
    pos_base1 = jnp.sum((pstart + exc) * occ1, axis=1)
    pos_base2 = jnp.sum((pstart + exc + occ1) * occ2, axis=1)
    p0_ref[...] = pos_base1
    p1_ref[...] = pos_base2

    nact = jnp.sum(padded) // _BLK
    bidx = lax.broadcasted_iota(jnp.int32, (_NB,), 0)
    brow = jnp.minimum(bidx, nact - 1)
    bexp = jnp.sum(
        (cum_p <= brow[:, None] * _BLK).astype(jnp.int32), axis=1)
    brow_ref[...] = brow
    bexp_ref[...] = jnp.clip(bexp, 0, _N_EXPERTS - 1)
    bact_ref[...] = (bidx < nact).astype(jnp.int32)


def _routing(hidden_states, gate_w):
    return pl.pallas_call(
        _routing_body,
        out_shape=[
            jax.ShapeDtypeStruct((_N_TOKENS,), jnp.float32),
            jax.ShapeDtypeStruct((_N_TOKENS,), jnp.float32),
            jax.ShapeDtypeStruct((_N_TOKENS,), jnp.int32),
            jax.ShapeDtypeStruct((_N_TOKENS,), jnp.int32),
            jax.ShapeDtypeStruct((_NB,), jnp.int32),
            jax.ShapeDtypeStruct((_NB,), jnp.int32),
            jax.ShapeDtypeStruct((_NB,), jnp.int32),
        ],
    )(hidden_states, gate_w)


_TOKW = _N_TOKENS // _NW


def _dispatch_body(hid_hbm, p0_hbm, p1_hbm, x_hbm, i0_v, i1_v, rows_v,
                   lsem, sem0, sem1):
    wid = lax.axis_index("s") * 2 + lax.axis_index("c")
    base = wid * _TOKW
    lr = pltpu.make_async_copy(hid_hbm.at[pl.ds(base, _TOKW)], rows_v, lsem)
    li0 = pltpu.make_async_copy(p0_hbm.at[wid], i0_v, sem0)
    li1 = pltpu.make_async_copy(p1_hbm.at[wid], i1_v, sem1)
    lr.start()
    li0.start()
    li1.start()
    li0.wait()
    li1.wait()
    lr.wait()
    c0 = pltpu.make_async_copy(rows_v, x_hbm.at[i0_v], sem0)
    c1 = pltpu.make_async_copy(rows_v, x_hbm.at[i1_v], sem1)
    c0.start()
    c1.start()
    c0.wait()
    c1.wait()


def _dispatch(hidden_states, p0r, p1r):
    mesh = plsc.VectorSubcoreMesh(core_axis_name="c", subcore_axis_name="s")
    return pl.kernel(
        _dispatch_body,
        out_type=jax.ShapeDtypeStruct((_P, _HIDDEN), jnp.float32),
        mesh=mesh,
        scratch_types=[
            pltpu.VMEM((_TOKW,), jnp.int32),
            pltpu.VMEM((_TOKW,), jnp.int32),
            pltpu.VMEM((_TOKW, _HIDDEN), jnp.float32),
            pltpu.SemaphoreType.DMA,
            pltpu.SemaphoreType.DMA,
            pltpu.SemaphoreType.DMA,
        ],
    )(hidden_states, p0r, p1r)


def _ffn_body(brow, bexp, bact, x_ref, wg_ref, wu_ref, wd_ref, y_ref):
    i = pl.program_id(0)

    @pl.when(bact[i] == 1)
    def _():
        x = x_ref[...]
        g = jnp.dot(x, wg_ref[0], preferred_element_type=jnp.float32)
        u = jnp.dot(x, wu_ref[0], preferred_element_type=jnp.float32)
        inter = (g * jax.nn.sigmoid(g)) * u
        y_ref[...] = jnp.dot(inter, wd_ref[0], preferred_element_type=jnp.float32)


def _ffn(x_sorted, w_gate, w_up, w_down, brow, bexp, bact):
    grid_spec = pltpu.PrefetchScalarGridSpec(
        num_scalar_prefetch=3,
        grid=(_NB,),
        in_specs=[
            pl.BlockSpec((_BLK, _HIDDEN), lambda i, br, be, ba: (br[i], 0)),
            pl.BlockSpec((1, _HIDDEN, _D_FF), lambda i, br, be, ba: (be[i], 0, 0)),
            pl.BlockSpec((1, _HIDDEN, _D_FF), lambda i, br, be, ba: (be[i], 0, 0)),
            pl.BlockSpec((1, _D_FF, _HIDDEN), lambda i, br, be, ba: (be[i], 0, 0)),
        ],
        out_specs=pl.BlockSpec((_BLK, _HIDDEN), lambda i, br, be, ba: (br[i], 0)),
    )
    return pl.pallas_call(
        _ffn_body,
        grid_spec=grid_spec,
        out_shape=jax.ShapeDtypeStruct((_P, _HIDDEN), jnp.float32),
    )(brow, bexp, bact, x_sorted, w_gate, w_up, w_down)


def _lane_bcast(vec, t):
    lane = jnp.full((_LANES, 1), t, jnp.int32)
    return lax.gather(
        vec, lane,
        lax.GatherDimensionNumbers(
            offset_dims=(), collapsed_slice_dims=(0,), start_index_map=(0,)),
        slice_sizes=(1,),
        mode=lax.GatherScatterMode.PROMISE_IN_BOUNDS)


_HTOK = _TOKW // 2


def _combine_body(y_hbm, p0_hbm, p1_hbm, w0_hbm, w1_hbm, out_hbm,
                  i0_v, i1_v, w0_v, w1_v, r0_v, r1_v, isem, wsem, gsem):
    wid = lax.axis_index("s") * 2 + lax.axis_index("c")
    base = wid * _TOKW
    li0 = pltpu.make_async_copy(p0_hbm.at[wid], i0_v, isem)
    li1 = pltpu.make_async_copy(p1_hbm.at[wid], i1_v, isem)
    lw0 = pltpu.make_async_copy(w0_hbm.at[wid], w0_v, wsem)
    lw1 = pltpu.make_async_copy(w1_hbm.at[wid], w1_v, wsem)
    li0.start()
    li1.start()
    lw0.start()
    lw1.start()
    li0.wait()
    li1.wait()
    g0 = pltpu.make_async_copy(y_hbm.at[i0_v], r0_v, gsem)
    g1 = pltpu.make_async_copy(y_hbm.at[i1_v], r1_v, gsem)
    g0.start()
    g1.start()
    lw0.wait()
    lw1.wait()
    g0.wait()
    g1.wait()

    def wadd_group(g, _):
        w0c = w0_v[pl.ds(g * _LANES, _LANES)]
        w1c = w1_v[pl.ds(g * _LANES, _LANES)]
        for t in range(_LANES):
            tok = g * _LANES + t
            w0b = _lane_bcast(w0c, t)
            w1b = _lane_bcast(w1c, t)
            for c in range(_HIDDEN // _LANES):
                sl = pl.ds(c * _LANES, _LANES)
                r0_v[tok, sl] = r0_v[tok, sl] * w0b + r1_v[tok, sl] * w1b
        return 0

    lax.fori_loop(0, _TOKW // _LANES, wadd_group, 0)
    pltpu.sync_copy(r0_v, out_hbm.at[pl.ds(base, _TOKW)])


def _combine(y_sorted, p0r, p1r, w0r, w1r):
    mesh = plsc.VectorSubcoreMesh(core_axis_name="c", subcore_axis_name="s")
    return pl.kernel(
        _combine_body,
        out_type=jax.ShapeDtypeStruct((_N_TOKENS, _HIDDEN), jnp.float32),
        mesh=mesh,
        scratch_types=[
            pltpu.VMEM((_TOKW,), jnp.int32),
            pltpu.VMEM((_TOKW,), jnp.int32),
            pltpu.VMEM((_TOKW,), jnp.float32),
            pltpu.VMEM((_TOKW,), jnp.float32),
            pltpu.VMEM((_TOKW, _HIDDEN), jnp.float32),
            pltpu.VMEM((_TOKW, _HIDDEN), jnp.float32),
            pltpu.SemaphoreType.DMA,
            pltpu.SemaphoreType.DMA,
            pltpu.SemaphoreType.DMA,
        ],
    )(y_sorted, p0r, p1r, w0r, w1r)


@jax.jit
def kernel(hidden_states, gate_w, w_gate, w_up, w_down):
    w1, w2, p0, p1, brow, bexp, bact = _routing(hidden_states, gate_w)

    p0r = p0.reshape(_NW, _TOKW)
    p1r = p1.reshape(_NW, _TOKW)
    w0r = w1.reshape(_NW, _TOKW)
    w1r = w2.reshape(_NW, _TOKW)

    x_sorted = _dispatch(hidden_states, p0r, p1r)
    y_sorted = _ffn(x_sorted, w_gate, w_up, w_down, brow, bexp, bact)
    return _combine(y_sorted, p0r, p1r, w0r, w1r)
